# Initial kernel scaffold; baseline (speedup 1.0000x reference)
import jax
import jax.numpy as jnp
from jax.experimental import pallas as pl


def kernel(x, intrinsics, pose):
    raise NotImplementedError("write your pallas kernel here")


# speedup vs baseline: 3.5215x; 3.5215x over previous
import functools

import jax
import jax.numpy as jnp
from jax import lax
from jax.experimental import pallas as pl
from jax.experimental.pallas import tpu as pltpu
from jax.experimental.pallas import tpu_sc as plsc

B, N, D, H, W, C = 2, 6, 41, 16, 44, 64
P = B * N * D * H * W
HALF = P // B
NX = 200
NCELL = NX * NX
ACC_ROWS = NCELL + 16
ROWS_PER_TILE = ACC_ROWS // 16
CHUNK_ROWS = 11
CHUNK_PTS = CHUNK_ROWS * 128
NCHUNK = HALF // CHUNK_PTS
CHALF = C // 2


def _cells(intrinsics, pose):
    ds_vals = jnp.arange(4.0, 45.0, 1.0, dtype=jnp.float32)
    ogf_h, ogf_w = H * 16, W * 16
    ones = jnp.ones((D, H, W), jnp.float32)
    ds_b = ds_vals.reshape(D, 1, 1) * ones
    xs = jnp.linspace(0.0, ogf_w - 1.0, W, dtype=jnp.float32).reshape(1, 1, W) * ones
    ys = jnp.linspace(0.0, ogf_h - 1.0, H, dtype=jnp.float32).reshape(1, H, 1) * ones
    frustum = jnp.stack((xs, ys, ds_b), -1)
    rots = pose[..., :3, :3]
    trans = pose[..., :3, 3]
    pts = jnp.concatenate(
        [frustum[..., :2] * frustum[..., 2:3], frustum[..., 2:3]], -1)
    combine = rots @ jnp.linalg.inv(intrinsics)
    geom = (jnp.einsum('bnij,dhwj->bndhwi', combine, pts)
            + trans[:, :, None, None, None, :])
    gf = jax.lax.stop_gradient(geom.reshape(P, 3))
    gx = gf[:, 0] * 2.0 + 100.0
    gy = gf[:, 1] * 2.0 + 100.0
    gz = (gf[:, 2] - 0.0 + 20.0 / 2.0) / 20.0
    ix = gx.astype(jnp.int32)
    iy = gy.astype(jnp.int32)
    iz = gz.astype(jnp.int32)
    kept = ((ix >= 0) & (ix < NX) & (iy >= 0) & (iy < NX)
            & (iz >= 0) & (iz < 1))
    return jnp.where(kept, ix * NX + iy, NCELL)


def _scatter_body(x2_hbm, cellr_hbm, zeros_hbm, out_hbm, data_v, idx_v, acc_sh):
    core = lax.axis_index("c")
    sub = lax.axis_index("s")
    zr = sub * ROWS_PER_TILE
    for p in range(2):
        pltpu.sync_copy(zeros_hbm.at[pl.ds(zr, ROWS_PER_TILE)],
                        acc_sh.at[pl.ds(zr, ROWS_PER_TILE)])
        plsc.subcore_barrier()

        @pl.loop(sub, NCHUNK, step=16)
        def _chunk(ci):
            base_pt = core * HALF + ci * CHUNK_PTS
            base_row = core * (HALF // 128) + ci * CHUNK_ROWS
            pltpu.sync_copy(cellr_hbm.at[pl.ds(base_row, CHUNK_ROWS)], idx_v)
            pltpu.sync_copy(
                x2_hbm.at[pl.ds(base_pt, CHUNK_PTS), pl.ds(p * CHALF, CHALF)],
                data_v)
            for j in range(CHUNK_ROWS):
                pltpu.sync_copy(data_v.at[pl.ds(j * 128, 128)],
                                acc_sh.at[idx_v.at[j]], add=True)

        plsc.subcore_barrier()
        pltpu.sync_copy(acc_sh.at[pl.ds(zr, ROWS_PER_TILE)],
                        out_hbm.at[core, p, pl.ds(zr, ROWS_PER_TILE)])
        plsc.subcore_barrier()


_scatter_call = functools.partial(
    pl.kernel,
    out_type=jax.ShapeDtypeStruct((B, 2, ACC_ROWS, CHALF), jnp.float32),
    mesh=plsc.VectorSubcoreMesh(core_axis_name="c", subcore_axis_name="s"),
    scratch_types=[
        pltpu.VMEM((CHUNK_PTS, CHALF), jnp.float32),
        pltpu.VMEM((CHUNK_ROWS, 128), jnp.int32),
        pltpu.VMEM_SHARED((ACC_ROWS, CHALF), jnp.float32),
    ],
    compiler_params=pltpu.CompilerParams(use_tc_tiling_on_sc=False),
)(_scatter_body)


def kernel(x, intrinsics, pose):
    cell = _cells(intrinsics, pose)
    cellr = cell.reshape(P // 128, 128)

    x2 = x.reshape(P, C)
    zeros = jnp.zeros((ACC_ROWS, CHALF), jnp.float32)
    acc = _scatter_call(x2, cellr, zeros)

    o = acc[:, :, :NCELL, :].reshape(B, 2, NX, NX, CHALF)
    return o.transpose(0, 1, 4, 3, 2).reshape(B, C, NX, NX)


# speedup vs baseline: 3.7172x; 1.0556x over previous
import functools

import jax
import jax.numpy as jnp
from jax import lax
from jax.experimental import pallas as pl
from jax.experimental.pallas import tpu as pltpu
from jax.experimental.pallas import tpu_sc as plsc

B, N, D, H, W, C = 2, 6, 41, 16, 44, 64
P = B * N * D * H * W
HW = H * W
NCHUNK = N * D
PAD_ROWS = 6
NX = 200
NCELL = NX * NX
ACC_ROWS = NCELL + 16
ZERO_PER_TILE = ACC_ROWS // 16
OUT_PER_TILE = NCELL // 16
CHALF = C // 2


def _cells(intrinsics, pose):
    ds_vals = jnp.arange(4.0, 45.0, 1.0, dtype=jnp.float32)
    ogf_h, ogf_w = H * 16, W * 16
    ones = jnp.ones((D, H, W), jnp.float32)
    ds_b = ds_vals.reshape(D, 1, 1) * ones
    xs = jnp.linspace(0.0, ogf_w - 1.0, W, dtype=jnp.float32).reshape(1, 1, W) * ones
    ys = jnp.linspace(0.0, ogf_h - 1.0, H, dtype=jnp.float32).reshape(1, H, 1) * ones
    frustum = jnp.stack((xs, ys, ds_b), -1)
    rots = pose[..., :3, :3]
    trans = pose[..., :3, 3]
    pts = jnp.concatenate(
        [frustum[..., :2] * frustum[..., 2:3], frustum[..., 2:3]], -1)
    combine = rots @ jnp.linalg.inv(intrinsics)
    geom = (jnp.einsum('bnij,dhwj->bndhwi', combine, pts)
            + trans[:, :, None, None, None, :])
    gf = jax.lax.stop_gradient(geom.reshape(P, 3))
    gx = gf[:, 0] * 2.0 + 100.0
    gy = gf[:, 1] * 2.0 + 100.0
    gz = (gf[:, 2] - 0.0 + 20.0 / 2.0) / 20.0
    ix = gx.astype(jnp.int32)
    iy = gy.astype(jnp.int32)
    iz = gz.astype(jnp.int32)
    kept = ((ix >= 0) & (ix < NX) & (iy >= 0) & (iy < NX)
            & (iz >= 0) & (iz < 1))
    return jnp.where(kept, ix * NX + iy, NCELL)


def _scatter_body(x5_hbm, cellp_hbm, zeros_hbm, out_hbm, data_v, idx_v, acc_sh):
    core = lax.axis_index("c")
    sub = lax.axis_index("s")
    for p in range(2):
        zr = sub * ZERO_PER_TILE
        pltpu.sync_copy(zeros_hbm.at[pl.ds(zr, ZERO_PER_TILE)],
                        acc_sh.at[pl.ds(zr, ZERO_PER_TILE)])
        plsc.subcore_barrier()

        @pl.loop(sub, NCHUNK, step=16)
        def _chunk(ci):
            n = ci // D
            d = ci - n * D
            pltpu.sync_copy(cellp_hbm.at[core * NCHUNK + ci], idx_v)
            pltpu.sync_copy(
                x5_hbm.at[core, n, d, pl.ds(0, HW), pl.ds(p * CHALF, CHALF)],
                data_v.at[pl.ds(0, HW)])
            for j in range(PAD_ROWS):
                pltpu.sync_copy(data_v.at[pl.ds(j * 128, 128)],
                                acc_sh.at[idx_v.at[j]], add=True)

        plsc.subcore_barrier()
        orow = sub * OUT_PER_TILE
        pltpu.sync_copy(acc_sh.at[pl.ds(orow, OUT_PER_TILE)],
                        out_hbm.at[core, p, pl.ds(orow, OUT_PER_TILE)])
        plsc.subcore_barrier()


_scatter_call = functools.partial(
    pl.kernel,
    out_type=jax.ShapeDtypeStruct((B, 2, NCELL, CHALF), jnp.float32),
    mesh=plsc.VectorSubcoreMesh(core_axis_name="c", subcore_axis_name="s"),
    scratch_types=[
        pltpu.VMEM((PAD_ROWS * 128, CHALF), jnp.float32),
        pltpu.VMEM((PAD_ROWS, 128), jnp.int32),
        pltpu.VMEM_SHARED((ACC_ROWS, CHALF), jnp.float32),
    ],
    compiler_params=pltpu.CompilerParams(use_tc_tiling_on_sc=False),
)(_scatter_body)


def kernel(x, intrinsics, pose):
    cell = _cells(intrinsics, pose)
    cellp = jnp.concatenate(
        [cell.reshape(B * N * D, HW),
         jnp.full((B * N * D, PAD_ROWS * 128 - HW), NCELL, jnp.int32)],
        axis=1).reshape(B * N * D, PAD_ROWS, 128)

    x5 = x.reshape(B, N, D, HW, C)
    zeros = jnp.zeros((ACC_ROWS, CHALF), jnp.float32)
    acc = _scatter_call(x5, cellp, zeros)

    o = acc.reshape(B, 2, NX, NX, CHALF)
    return o.transpose(0, 1, 4, 3, 2).reshape(B, C, NX, NX)


# speedup vs baseline: 3.7176x; 1.0001x over previous
import functools

import jax
import jax.numpy as jnp
from jax import lax
from jax.experimental import pallas as pl
from jax.experimental.pallas import tpu as pltpu
from jax.experimental.pallas import tpu_sc as plsc

B, N, D, H, W, C = 2, 6, 41, 16, 44, 64
P = B * N * D * H * W
HW = H * W
NCHUNK = N * D
PAD_ROWS = 6
NX = 200
NCELL = NX * NX
ACC_ROWS = NCELL + 16
ZERO_PER_TILE = ACC_ROWS // 16
OUT_PER_TILE = NCELL // 16
CHALF = C // 2


def _cells(intrinsics, pose):
    ds_vals = jnp.arange(4.0, 45.0, 1.0, dtype=jnp.float32)
    ogf_h, ogf_w = H * 16, W * 16
    ones = jnp.ones((D, H, W), jnp.float32)
    ds_b = ds_vals.reshape(D, 1, 1) * ones
    xs = jnp.linspace(0.0, ogf_w - 1.0, W, dtype=jnp.float32).reshape(1, 1, W) * ones
    ys = jnp.linspace(0.0, ogf_h - 1.0, H, dtype=jnp.float32).reshape(1, H, 1) * ones
    frustum = jnp.stack((xs, ys, ds_b), -1)
    rots = pose[..., :3, :3]
    trans = pose[..., :3, 3]
    pts = jnp.concatenate(
        [frustum[..., :2] * frustum[..., 2:3], frustum[..., 2:3]], -1)
    combine = rots @ jnp.linalg.inv(intrinsics)
    geom = (jnp.einsum('bnij,dhwj->bndhwi', combine, pts)
            + trans[:, :, None, None, None, :])
    gf = jax.lax.stop_gradient(geom.reshape(P, 3))
    gx = gf[:, 0] * 2.0 + 100.0
    gy = gf[:, 1] * 2.0 + 100.0
    gz = (gf[:, 2] - 0.0 + 20.0 / 2.0) / 20.0
    ix = gx.astype(jnp.int32)
    iy = gy.astype(jnp.int32)
    iz = gz.astype(jnp.int32)
    kept = ((ix >= 0) & (ix < NX) & (iy >= 0) & (iy < NX)
            & (iz >= 0) & (iz < 1))
    return jnp.where(kept, ix * NX + iy, NCELL)


SLOTS = 16


def _scatter_body(x_hbm, cellp_hbm, zeros_hbm, out_hbm, data_v, idx_v, acc_sh,
                  sem_li0, sem_li1, sem_ld0, sem_ld1, sem_sc0, sem_sc1):
    core = lax.axis_index("c")
    sub = lax.axis_index("s")
    sem_li = (sem_li0, sem_li1)
    sem_ld = (sem_ld0, sem_ld1)
    sem_sc = (sem_sc0, sem_sc1)

    def issue_loads(s, par, p):
        raw = sub + 16 * s
        ci = jnp.minimum(raw, NCHUNK - 1)
        cp = jnp.where(raw < NCHUNK, core * NCHUNK + ci, 2 * NCHUNK)
        n = ci // D
        d = ci - n * D
        pltpu.async_copy(cellp_hbm.at[cp], idx_v.at[par], sem_li[par])
        pltpu.async_copy(
            x_hbm.at[core, n, d, pl.ds(0, HW), pl.ds(p * CHALF, CHALF)],
            data_v.at[par, pl.ds(0, HW)], sem_ld[par])

    def wait_loads(par, p):
        pltpu.make_async_copy(cellp_hbm.at[0], idx_v.at[par],
                              sem_li[par]).wait()
        pltpu.make_async_copy(
            x_hbm.at[core, 0, 0, pl.ds(0, HW), pl.ds(p * CHALF, CHALF)],
            data_v.at[par, pl.ds(0, HW)], sem_ld[par]).wait()

    def fire_scatters(par):
        for j in range(PAD_ROWS):
            pltpu.async_copy(data_v.at[par, pl.ds(j * 128, 128)],
                             acc_sh.at[idx_v.at[par, j]], sem_sc[par],
                             add=True)

    def drain_scatters(par):
        for j in range(PAD_ROWS):
            pltpu.make_async_copy(data_v.at[par, pl.ds(j * 128, 128)],
                                  acc_sh.at[idx_v.at[par, j]],
                                  sem_sc[par]).wait()

    for p in range(2):
        zr = sub * ZERO_PER_TILE
        pltpu.sync_copy(zeros_hbm.at[pl.ds(zr, ZERO_PER_TILE)],
                        acc_sh.at[pl.ds(zr, ZERO_PER_TILE)])
        plsc.subcore_barrier()

        issue_loads(0, 0, p)
        wait_loads(0, p)
        issue_loads(1, 1, p)
        fire_scatters(0)

        @pl.loop(0, (SLOTS - 2) // 2)
        def _kk(kk):
            s = 1 + 2 * kk
            for par in (1, 0):
                wait_loads(par, p)
                drain_scatters(1 - par)
                issue_loads(s + (1 if par == 1 else 2), 1 - par, p)
                fire_scatters(par)

        wait_loads(1, p)
        drain_scatters(0)
        fire_scatters(1)
        drain_scatters(1)

        plsc.subcore_barrier()
        orow = sub * OUT_PER_TILE
        pltpu.sync_copy(acc_sh.at[pl.ds(orow, OUT_PER_TILE)],
                        out_hbm.at[core, p, pl.ds(orow, OUT_PER_TILE)])
        plsc.subcore_barrier()


_scatter_call = functools.partial(
    pl.kernel,
    out_type=jax.ShapeDtypeStruct((B, 2, NCELL, CHALF), jnp.float32),
    mesh=plsc.VectorSubcoreMesh(core_axis_name="c", subcore_axis_name="s"),
    scratch_types=[
        pltpu.VMEM((2, PAD_ROWS * 128, CHALF), jnp.float32),
        pltpu.VMEM((2, PAD_ROWS, 128), jnp.int32),
        pltpu.VMEM_SHARED((ACC_ROWS, CHALF), jnp.float32),
        pltpu.SemaphoreType.DMA,
        pltpu.SemaphoreType.DMA,
        pltpu.SemaphoreType.DMA,
        pltpu.SemaphoreType.DMA,
        pltpu.SemaphoreType.DMA,
        pltpu.SemaphoreType.DMA,
    ],
    compiler_params=pltpu.CompilerParams(use_tc_tiling_on_sc=False),
)(_scatter_body)


def kernel(x, intrinsics, pose):
    cell = _cells(intrinsics, pose)
    cellp = jnp.concatenate(
        [cell.reshape(B * N * D, HW),
         jnp.full((B * N * D, PAD_ROWS * 128 - HW), NCELL, jnp.int32)],
        axis=1).reshape(B * N * D, PAD_ROWS, 128)
    cellp = jnp.concatenate(
        [cellp, jnp.full((1, PAD_ROWS, 128), NCELL, jnp.int32)], axis=0)

    zeros = jnp.zeros((ACC_ROWS, CHALF), jnp.float32)
    acc = _scatter_call(x.reshape(B, N, D, HW, C), cellp, zeros)

    o = acc.reshape(B, 2, NX, NX, CHALF)
    return o.transpose(0, 1, 4, 3, 2).reshape(B, C, NX, NX)
